# baseline (speedup 1.0000x reference)
import functools

import jax
import jax.numpy as jnp
from jax import lax
from jax.experimental import pallas as pl
from jax.experimental.pallas import tpu as pltpu
from jax.experimental.pallas import tpu_sc as plsc

N = 10000
E = 320000
D_IN = 128
D_OUT = 64
NT = 16
NSC = 2
NW = NT * NSC
EPT = E // NW
GRP = 80
NG = EPT // GRP
RPT = N // NT
DRW = 160
NBUF = 3


def _pre_body(head_ref, tail_ref, w_ref, al_ref, ar_ref, emb_ref, we_ref,
              ae_ref, wres_ref, bres_ref,
              htail_ref, hl_ref, hr_ref, he_ref, res_ref):
    w = w_ref[...]
    hh = jnp.dot(head_ref[...], w, preferred_element_type=jnp.float32)
    ht = jnp.dot(tail_ref[...], w, preferred_element_type=jnp.float32)
    htail_ref[...] = ht
    hl_ref[...] = jnp.dot(hh, al_ref[...], preferred_element_type=jnp.float32)
    hr_ref[...] = jnp.dot(ht, ar_ref[...], preferred_element_type=jnp.float32)
    e5 = jnp.dot(emb_ref[...], we_ref[...], preferred_element_type=jnp.float32)
    he5 = jnp.sum(e5 * ae_ref[...], axis=1, keepdims=True)
    he_ref[...] = jnp.concatenate(
        [he5, jnp.zeros((11, 1), jnp.float32)], axis=0)
    res_ref[...] = (
        jnp.dot(head_ref[...], wres_ref[...], preferred_element_type=jnp.float32)
        + bres_ref[...])


_pre_call = pl.pallas_call(
    _pre_body,
    out_shape=(
        jax.ShapeDtypeStruct((N, D_OUT), jnp.float32),
        jax.ShapeDtypeStruct((N, 1), jnp.float32),
        jax.ShapeDtypeStruct((N, 1), jnp.float32),
        jax.ShapeDtypeStruct((16, 1), jnp.float32),
        jax.ShapeDtypeStruct((N, D_OUT), jnp.float32),
    ),
)


_mesh = plsc.VectorSubcoreMesh(core_axis_name="c", subcore_axis_name="s")


@functools.partial(
    pl.kernel,
    out_type=(
        jax.ShapeDtypeStruct((NSC, N, D_OUT), jnp.float32),
        jax.ShapeDtypeStruct((NSC, DRW, 64), jnp.float32),
        jax.ShapeDtypeStruct((NSC, 8, 16), jnp.float32),
    ),
    mesh=_mesh,
    compiler_params=pltpu.CompilerParams(
        needs_layout_passes=False, use_tc_tiling_on_sc=False),
    scratch_types=(
        pltpu.VMEM((N,), jnp.float32),
        pltpu.VMEM((N,), jnp.float32),
        pltpu.VMEM((16,), jnp.float32),
        pltpu.VMEM((NG, GRP), jnp.int32),
        pltpu.VMEM((NG, GRP), jnp.int32),
        pltpu.VMEM((NG, GRP), jnp.int32),
        pltpu.VMEM((NG, GRP), jnp.float32),
        pltpu.VMEM((NBUF, GRP, D_OUT), jnp.float32),
        pltpu.VMEM((DRW, 64), jnp.float32),
        pltpu.VMEM((2, GRP), jnp.int32),
        pltpu.VMEM((16,), jnp.float32),
        pltpu.VMEM((16, 16), jnp.float32),
        pltpu.VMEM_SHARED((N, D_OUT), jnp.float32),
        pltpu.VMEM_SHARED((DRW, 64), jnp.float32),
        pltpu.VMEM_SHARED((16, 16), jnp.float32),
        pltpu.SemaphoreType.DMA,
        pltpu.SemaphoreType.DMA,
        pltpu.SemaphoreType.DMA,
        pltpu.SemaphoreType.DMA,
        pltpu.SemaphoreType.DMA,
        pltpu.SemaphoreType.DMA,
    ),
)
def _sc_kernel(hl_hbm, hr_hbm, he_hbm, head_hbm, tail_hbm, tmp_hbm, htail_hbm,
               acc_out, den_out, gmax_out,
               hl_t, hr_t, he_t, head_t, tail_t, tmp_t, ex_t, rows_t,
               den_t, iota_t, mbuf, gall,
               acc_sh, den_sh, gmax_sh,
               semg0, semg1, semg2, sems0, sems1, sems2):
    cid = lax.axis_index("c")
    sid = lax.axis_index("s")
    w = cid * NT + sid

    pltpu.sync_copy(hl_hbm, hl_t)
    pltpu.sync_copy(hr_hbm, hr_t)
    pltpu.sync_copy(he_hbm, he_t)
    pltpu.sync_copy(head_hbm.at[w], head_t)
    pltpu.sync_copy(tail_hbm.at[w], tail_t)
    pltpu.sync_copy(tmp_hbm.at[w], tmp_t)

    zv = jnp.zeros((16,), jnp.float32)
    lanes = lax.iota(jnp.int32, 16)

    def zden(i, c):
        for v in range(4):
            den_t[i, pl.ds(v * 16, 16)] = zv
        return c

    lax.fori_loop(0, DRW, zden, 0)
    for v in range(GRP // 16):
        iota_t[0, pl.ds(v * 16, 16)] = lanes + v * 16
        iota_t[1, pl.ds(v * 16, 16)] = lanes + (GRP + v * 16)

    for k in range(5):
        pltpu.sync_copy(den_t.at[pl.ds(0, 125)],
                        acc_sh.at[pl.ds(sid * RPT + k * 125, 125)])
    pltpu.sync_copy(den_t.at[pl.ds(0, DRW // NT)],
                    den_sh.at[pl.ds(sid * (DRW // NT), DRW // NT)])

    def p1(g, m):
        for v in range(GRP // 16):
            sl = pl.ds(v * 16, 16)
            hi = head_t[g, sl]
            ti = tail_t[g, sl]
            ei = tmp_t[g, sl]
            s = (plsc.load_gather(hl_t, [hi])
                 + plsc.load_gather(hr_t, [ti])
                 + plsc.load_gather(he_t, [ei]))
            s = jnp.where(s >= 0.0, s, 0.2 * s)
            ex_t[g, sl] = s
            m = jnp.maximum(m, s)
        return m

    m = lax.fori_loop(0, NG, p1, jnp.full((16,), -3.0e38, jnp.float32))

    mbuf[...] = jnp.full((16,), jnp.max(m), jnp.float32)
    pltpu.sync_copy(mbuf, gmax_sh.at[sid])
    plsc.subcore_barrier()
    pltpu.sync_copy(gmax_sh, gall)
    gv = gall[0, :]
    for t in range(1, 16):
        gv = jnp.maximum(gv, gall[t, :])
    mbuf[...] = gv

    @pl.when(sid == 0)
    def _():
        pltpu.sync_copy(mbuf, gmax_out.at[cid, 0])

    def p1c(g, c):
        for v in range(GRP // 16):
            sl = pl.ds(v * 16, 16)
            ex = jnp.exp(ex_t[g, sl] - gv)
            ex_t[g, sl] = ex
            hv = head_t[g, sl]
            hi = lax.shift_right_logical(hv, 6)
            lo = lax.bitwise_and(hv, 63)
            for j in range(16):
                plsc.addupdate_scatter(
                    den_t, [hi, lo], ex, mask=lanes == j)
        return c

    lax.fori_loop(0, NG, p1c, 0)

    for k in range(2):
        pltpu.sync_copy(den_t.at[pl.ds(k * GRP, GRP)],
                        den_sh.at[iota_t.at[k]], add=True)

    sems_g = (semg0, semg1, semg2)
    sems_s = (sems0, sems1, sems2)

    def start_gather(g, b):
        pltpu.async_copy(htail_hbm.at[tail_t.at[g]], rows_t.at[b], sems_g[b])

    def wait_gather(g, b):
        pltpu.make_async_copy(
            htail_hbm.at[tail_t.at[g]], rows_t.at[b], sems_g[b]).wait()

    def start_scatter(g, b):
        pltpu.async_copy(
            rows_t.at[b], acc_sh.at[head_t.at[g]], sems_s[b], add=True)

    def wait_scatter(g, b):
        pltpu.make_async_copy(
            rows_t.at[b], acc_sh.at[head_t.at[g]], sems_s[b]).wait()

    def scale(g, b):
        for k in range(GRP // 16):
            exg = ex_t[g, pl.ds(k * 16, 16)]
            sps = [
                exg.at[jnp.full((16,), j, jnp.int32)].get(
                    mode="promise_in_bounds")
                for j in range(16)
            ]
            for j in range(16):
                r = k * 16 + j
                for v in range(D_OUT // 16):
                    sl = pl.ds(v * 16, 16)
                    rows_t[b, r, sl] = rows_t[b, r, sl] * sps[j]

    def step(g, b, nb):
        wait_gather(g, b)
        scale(g, b)
        start_scatter(g, b)

        @pl.when(g + 2 < NG)
        def _():
            @pl.when(g >= 1)
            def _():
                wait_scatter(g - 1, nb)
            start_gather(g + 2, nb)

    start_gather(0, 0)
    start_gather(1, 1)

    def p2(g, c):
        gm = lax.rem(g, 3)
        for b in range(NBUF):
            @pl.when(gm == b)
            def _(b=b):
                step(g, b, (b + 2) % 3)
        return c

    lax.fori_loop(0, NG, p2, 0)
    for q in range(NBUF):
        gq = NG - NBUF + q
        wait_scatter(gq, gq % 3)

    plsc.subcore_barrier()
    pltpu.sync_copy(acc_sh.at[pl.ds(sid * RPT, RPT)],
                    acc_out.at[cid, pl.ds(sid * RPT, RPT)])
    pltpu.sync_copy(den_sh.at[pl.ds(sid * (DRW // NT), DRW // NT)],
                    den_out.at[cid, pl.ds(sid * (DRW // NT), DRW // NT)])


def _post_body(acc_ref, den_ref, s_ref, res_ref, out_ref):
    num = acc_ref[0] * s_ref[0] + acc_ref[1] * s_ref[1]
    den = den_ref[0, 0:N] * s_ref[0] + den_ref[1, 0:N] * s_ref[1]
    recip = jnp.where(den > 0.0, 1.0 / den, 0.0)
    out_ref[...] = num * recip + res_ref[...]


_post_call = pl.pallas_call(
    _post_body,
    in_specs=(
        pl.BlockSpec(memory_space=pltpu.VMEM),
        pl.BlockSpec(memory_space=pltpu.VMEM),
        pl.BlockSpec(memory_space=pltpu.SMEM),
        pl.BlockSpec(memory_space=pltpu.VMEM),
    ),
    out_shape=jax.ShapeDtypeStruct((N, D_OUT), jnp.float32),
)


def kernel(head_feature, tail_feature, edge_index, tmp_edge, W, W_e,
           a_l, a_r, a_e, edge_emb, W_res, b_res):
    al = a_l.reshape(D_OUT, 1)
    ar = a_r.reshape(D_OUT, 1)
    ae = a_e.reshape(1, 64)
    bres2 = b_res.reshape(1, D_OUT)

    htail, hl_out, hr_out, he_out, res = _pre_call(
        head_feature, tail_feature, W, al, ar, edge_emb, W_e, ae, W_res, bres2)

    hl = hl_out.reshape(N)
    hr = hr_out.reshape(N)
    he16 = he_out.reshape(16)
    head3 = edge_index[0].reshape(NW, NG, GRP)
    tail3 = edge_index[1].reshape(NW, NG, GRP)
    tmp3 = tmp_edge.reshape(NW, NG, GRP)

    acc, den, gmax = _sc_kernel(hl, hr, he16, head3, tail3, tmp3, htail)

    g = jnp.max(gmax[:, 0, :], axis=1)
    s2 = jnp.exp(g - jnp.max(g))
    denP = den.reshape(NSC, DRW * 64, 1)
    return _post_call(acc, denP, s2, res)


# speedup vs baseline: 1.0749x; 1.0749x over previous
import functools

import jax
import jax.numpy as jnp
from jax import lax
from jax.experimental import pallas as pl
from jax.experimental.pallas import tpu as pltpu
from jax.experimental.pallas import tpu_sc as plsc

N = 10000
E = 320000
D_IN = 128
D_OUT = 64
NT = 16
NSC = 2
NW = NT * NSC
EPT = E // NW
GRP = 80
NG = EPT // GRP
RPT = N // NT
DRW = 160
NBUF = 3


def _pre_body(head_ref, tail_ref, w_ref, al_ref, ar_ref, emb_ref, we_ref,
              ae_ref, wres_ref, bres_ref,
              htail_ref, hlr_ref, he_ref, res_ref):
    w = w_ref[...]
    hh = jnp.dot(head_ref[...], w, preferred_element_type=jnp.float32)
    ht = jnp.dot(tail_ref[...], w, preferred_element_type=jnp.float32)
    htail_ref[...] = ht
    hlr_ref[:, 0:1] = jnp.dot(hh, al_ref[...], preferred_element_type=jnp.float32)
    hlr_ref[:, 1:2] = jnp.dot(ht, ar_ref[...], preferred_element_type=jnp.float32)
    e5 = jnp.dot(emb_ref[...], we_ref[...], preferred_element_type=jnp.float32)
    he5 = jnp.sum(e5 * ae_ref[...], axis=1, keepdims=True)
    he_ref[...] = jnp.concatenate(
        [he5, jnp.zeros((11, 1), jnp.float32)], axis=0)
    res_ref[...] = (
        jnp.dot(head_ref[...], wres_ref[...], preferred_element_type=jnp.float32)
        + bres_ref[...])


_pre_call = pl.pallas_call(
    _pre_body,
    out_shape=(
        jax.ShapeDtypeStruct((N, D_OUT), jnp.float32),
        jax.ShapeDtypeStruct((N, 2), jnp.float32),
        jax.ShapeDtypeStruct((16, 1), jnp.float32),
        jax.ShapeDtypeStruct((N, D_OUT), jnp.float32),
    ),
)


_mesh = plsc.VectorSubcoreMesh(core_axis_name="c", subcore_axis_name="s")


@functools.partial(
    pl.kernel,
    out_type=(
        jax.ShapeDtypeStruct((NSC, N, D_OUT), jnp.float32),
        jax.ShapeDtypeStruct((NSC, DRW, 64), jnp.float32),
        jax.ShapeDtypeStruct((NSC, 8, 16), jnp.float32),
    ),
    mesh=_mesh,
    compiler_params=pltpu.CompilerParams(
        needs_layout_passes=False, use_tc_tiling_on_sc=False),
    scratch_types=(
        pltpu.VMEM((N,), jnp.float32),
        pltpu.VMEM((N,), jnp.float32),
        pltpu.VMEM((16,), jnp.float32),
        pltpu.VMEM((EPT,), jnp.int32),
        pltpu.VMEM((EPT,), jnp.int32),
        pltpu.VMEM((EPT,), jnp.int32),
        pltpu.VMEM((EPT,), jnp.float32),
        pltpu.VMEM((NBUF, GRP, D_OUT), jnp.float32),
        pltpu.VMEM((DRW, 64), jnp.float32),
        pltpu.VMEM((2, GRP), jnp.int32),
        pltpu.VMEM((16,), jnp.float32),
        pltpu.VMEM((16, 16), jnp.float32),
        pltpu.VMEM_SHARED((N, D_OUT), jnp.float32),
        pltpu.VMEM_SHARED((DRW, 64), jnp.float32),
        pltpu.VMEM_SHARED((16, 16), jnp.float32),
        pltpu.SemaphoreType.DMA,
        pltpu.SemaphoreType.DMA,
        pltpu.SemaphoreType.DMA,
        pltpu.SemaphoreType.DMA,
        pltpu.SemaphoreType.DMA,
        pltpu.SemaphoreType.DMA,
    ),
)
def _sc_kernel(hl_hbm, hr_hbm, he_hbm, edge_hbm, tmp_hbm, htail_hbm,
               acc_out, den_out, gmax_out,
               hl_t, hr_t, he_t, head_t, tail_t, tmp_t, ex_t, rows_t,
               den_t, iota_t, mbuf, gall,
               acc_sh, den_sh, gmax_sh,
               semg0, semg1, semg2, sems0, sems1, sems2):
    cid = lax.axis_index("c")
    sid = lax.axis_index("s")
    w = cid * NT + sid

    pltpu.sync_copy(hl_hbm, hl_t)
    pltpu.sync_copy(hr_hbm, hr_t)
    pltpu.sync_copy(he_hbm, he_t)
    pltpu.sync_copy(edge_hbm.at[0, pl.ds(w * EPT, EPT)], head_t)
    pltpu.sync_copy(edge_hbm.at[1, pl.ds(w * EPT, EPT)], tail_t)
    pltpu.sync_copy(tmp_hbm.at[pl.ds(w * EPT, EPT)], tmp_t)

    zv = jnp.zeros((16,), jnp.float32)
    lanes = lax.iota(jnp.int32, 16)

    def zden(i, c):
        for v in range(4):
            den_t[i, pl.ds(v * 16, 16)] = zv
        return c

    lax.fori_loop(0, DRW, zden, 0)
    for v in range(GRP // 16):
        iota_t[0, pl.ds(v * 16, 16)] = lanes + v * 16
        iota_t[1, pl.ds(v * 16, 16)] = lanes + (GRP + v * 16)

    for k in range(5):
        pltpu.sync_copy(den_t.at[pl.ds(0, 125)],
                        acc_sh.at[pl.ds(sid * RPT + k * 125, 125)])
    pltpu.sync_copy(den_t.at[pl.ds(0, DRW // NT)],
                    den_sh.at[pl.ds(sid * (DRW // NT), DRW // NT)])

    def p1(g, m):
        for v in range(GRP // 16):
            sl = pl.ds(g * GRP + v * 16, 16)
            hi = head_t[sl]
            ti = tail_t[sl]
            ei = tmp_t[sl]
            s = (plsc.load_gather(hl_t, [hi])
                 + plsc.load_gather(hr_t, [ti])
                 + plsc.load_gather(he_t, [ei]))
            s = jnp.where(s >= 0.0, s, 0.2 * s)
            ex_t[sl] = s
            m = jnp.maximum(m, s)
        return m

    m = lax.fori_loop(0, NG, p1, jnp.full((16,), -3.0e38, jnp.float32))

    mbuf[...] = jnp.full((16,), jnp.max(m), jnp.float32)
    pltpu.sync_copy(mbuf, gmax_sh.at[sid])
    plsc.subcore_barrier()
    pltpu.sync_copy(gmax_sh, gall)
    gv = gall[0, :]
    for t in range(1, 16):
        gv = jnp.maximum(gv, gall[t, :])
    mbuf[...] = gv

    @pl.when(sid == 0)
    def _():
        pltpu.sync_copy(mbuf, gmax_out.at[cid, 0])

    def p1c(g, c):
        for v in range(GRP // 16):
            sl = pl.ds(g * GRP + v * 16, 16)
            ex = jnp.exp(ex_t[sl] - gv)
            ex_t[sl] = ex
            hv = head_t[sl]
            hi = lax.shift_right_logical(hv, 6)
            lo = lax.bitwise_and(hv, 63)
            for j in range(16):
                plsc.addupdate_scatter(
                    den_t, [hi, lo], ex, mask=lanes == j)
        return c

    lax.fori_loop(0, NG, p1c, 0)

    for k in range(2):
        pltpu.sync_copy(den_t.at[pl.ds(k * GRP, GRP)],
                        den_sh.at[iota_t.at[k]], add=True)

    sems_g = (semg0, semg1, semg2)
    sems_s = (sems0, sems1, sems2)

    def start_gather(g, b):
        pltpu.async_copy(htail_hbm.at[tail_t.at[pl.ds(g * GRP, GRP)]],
                         rows_t.at[b], sems_g[b])

    def wait_gather(g, b):
        pltpu.make_async_copy(
            htail_hbm.at[tail_t.at[pl.ds(g * GRP, GRP)]],
            rows_t.at[b], sems_g[b]).wait()

    def start_scatter(g, b):
        pltpu.async_copy(
            rows_t.at[b], acc_sh.at[head_t.at[pl.ds(g * GRP, GRP)]],
            sems_s[b], add=True)

    def wait_scatter(g, b):
        pltpu.make_async_copy(
            rows_t.at[b], acc_sh.at[head_t.at[pl.ds(g * GRP, GRP)]],
            sems_s[b]).wait()

    def scale(g, b):
        for k in range(GRP // 16):
            exg = ex_t[pl.ds(g * GRP + k * 16, 16)]
            sps = [
                exg.at[jnp.full((16,), j, jnp.int32)].get(
                    mode="promise_in_bounds")
                for j in range(16)
            ]
            for j in range(16):
                r = k * 16 + j
                for v in range(D_OUT // 16):
                    sl = pl.ds(v * 16, 16)
                    rows_t[b, r, sl] = rows_t[b, r, sl] * sps[j]

    def step(g, b, nb):
        wait_gather(g, b)
        scale(g, b)
        start_scatter(g, b)

        @pl.when(g + 2 < NG)
        def _():
            @pl.when(g >= 1)
            def _():
                wait_scatter(g - 1, nb)
            start_gather(g + 2, nb)

    start_gather(0, 0)
    start_gather(1, 1)

    def p2(g, c):
        gm = lax.rem(g, 3)
        for b in range(NBUF):
            @pl.when(gm == b)
            def _(b=b):
                step(g, b, (b + 2) % 3)
        return c

    lax.fori_loop(0, NG, p2, 0)
    for q in range(NBUF):
        gq = NG - NBUF + q
        wait_scatter(gq, gq % 3)

    plsc.subcore_barrier()
    pltpu.sync_copy(acc_sh.at[pl.ds(sid * RPT, RPT)],
                    acc_out.at[cid, pl.ds(sid * RPT, RPT)])
    pltpu.sync_copy(den_sh.at[pl.ds(sid * (DRW // NT), DRW // NT)],
                    den_out.at[cid, pl.ds(sid * (DRW // NT), DRW // NT)])


def _post_body(acc_ref, den_ref, s_ref, res_ref, out_ref):
    num = acc_ref[0] * s_ref[0] + acc_ref[1] * s_ref[1]
    den = den_ref[0, 0:N] * s_ref[0] + den_ref[1, 0:N] * s_ref[1]
    recip = jnp.where(den > 0.0, 1.0 / den, 0.0)
    out_ref[...] = num * recip + res_ref[...]


_post_call = pl.pallas_call(
    _post_body,
    in_specs=(
        pl.BlockSpec(memory_space=pltpu.VMEM),
        pl.BlockSpec(memory_space=pltpu.VMEM),
        pl.BlockSpec(memory_space=pltpu.SMEM),
        pl.BlockSpec(memory_space=pltpu.VMEM),
    ),
    out_shape=jax.ShapeDtypeStruct((N, D_OUT), jnp.float32),
)


def kernel(head_feature, tail_feature, edge_index, tmp_edge, W, W_e,
           a_l, a_r, a_e, edge_emb, W_res, b_res):
    al = a_l.reshape(D_OUT, 1)
    ar = a_r.reshape(D_OUT, 1)
    ae = a_e.reshape(1, 64)
    bres2 = b_res.reshape(1, D_OUT)

    htail, hlr, he_out, res = _pre_call(
        head_feature, tail_feature, W, al, ar, edge_emb, W_e, ae, W_res, bres2)

    hl = hlr[:, 0]
    hr = hlr[:, 1]
    he16 = he_out.reshape(16)

    acc, den, gmax = _sc_kernel(hl, hr, he16, edge_index, tmp_edge, htail)

    g = jnp.max(gmax[:, 0, :], axis=1)
    s2 = jnp.exp(g - jnp.max(g))
    denP = den.reshape(NSC, DRW * 64, 1)
    return _post_call(acc, denP, s2, res)


# speedup vs baseline: 1.1088x; 1.0315x over previous
import functools

import jax
import jax.numpy as jnp
from jax import lax
from jax.experimental import pallas as pl
from jax.experimental.pallas import tpu as pltpu
from jax.experimental.pallas import tpu_sc as plsc

N = 10000
E = 320000
D_IN = 128
D_OUT = 64
NT = 16
NSC = 2
NW = NT * NSC
EPT = E // NW
GRP = 80
NG = EPT // GRP
RPT = N // NT
DRW = 160
NBUF = 3


def _pre_body(head_ref, tail_ref, w_ref, al_ref, ar_ref, emb_ref, we_ref,
              ae_ref, wres_ref, bres_ref,
              htail_ref, hlr_ref, he_ref, res_ref):
    w = w_ref[...]
    hh = jnp.dot(head_ref[...], w, preferred_element_type=jnp.float32)
    ht = jnp.dot(tail_ref[...], w, preferred_element_type=jnp.float32)
    htail_ref[...] = ht
    hlr_ref[:, 0:1] = jnp.dot(hh, al_ref[...], preferred_element_type=jnp.float32)
    hlr_ref[:, 1:2] = jnp.dot(ht, ar_ref[...], preferred_element_type=jnp.float32)
    e5 = jnp.dot(emb_ref[...], we_ref[...], preferred_element_type=jnp.float32)
    he5 = jnp.sum(e5 * ae_ref[...], axis=1, keepdims=True)
    he_ref[...] = jnp.concatenate(
        [he5, jnp.zeros((11, 1), jnp.float32)], axis=0)
    res_ref[...] = (
        jnp.dot(head_ref[...], wres_ref[...], preferred_element_type=jnp.float32)
        + bres_ref[...])


_pre_call = pl.pallas_call(
    _pre_body,
    out_shape=(
        jax.ShapeDtypeStruct((N, D_OUT), jnp.float32),
        jax.ShapeDtypeStruct((N, 2), jnp.float32),
        jax.ShapeDtypeStruct((16, 1), jnp.float32),
        jax.ShapeDtypeStruct((N, D_OUT), jnp.float32),
    ),
)


_mesh = plsc.VectorSubcoreMesh(core_axis_name="c", subcore_axis_name="s")


@functools.partial(
    pl.kernel,
    out_type=(
        jax.ShapeDtypeStruct((NSC, N, D_OUT), jnp.float32),
        jax.ShapeDtypeStruct((NSC, DRW, 64), jnp.float32),
        jax.ShapeDtypeStruct((NSC, 8, 16), jnp.float32),
    ),
    mesh=_mesh,
    compiler_params=pltpu.CompilerParams(
        needs_layout_passes=False, use_tc_tiling_on_sc=False),
    scratch_types=(
        pltpu.VMEM((N,), jnp.float32),
        pltpu.VMEM((N,), jnp.float32),
        pltpu.VMEM((16,), jnp.float32),
        pltpu.VMEM((EPT,), jnp.int32),
        pltpu.VMEM((EPT,), jnp.int32),
        pltpu.VMEM((EPT,), jnp.int32),
        pltpu.VMEM((EPT,), jnp.float32),
        pltpu.VMEM((NBUF, GRP, D_OUT), jnp.float32),
        pltpu.VMEM((DRW, 64), jnp.float32),
        pltpu.VMEM((2, GRP), jnp.int32),
        pltpu.VMEM((16,), jnp.float32),
        pltpu.VMEM((16, 16), jnp.float32),
        pltpu.VMEM_SHARED((N, D_OUT), jnp.float32),
        pltpu.VMEM_SHARED((DRW, 64), jnp.float32),
        pltpu.VMEM_SHARED((16, 16), jnp.float32),
        pltpu.SemaphoreType.DMA,
        pltpu.SemaphoreType.DMA,
        pltpu.SemaphoreType.DMA,
        pltpu.SemaphoreType.DMA,
        pltpu.SemaphoreType.DMA,
        pltpu.SemaphoreType.DMA,
    ),
)
def _sc_kernel(hl_hbm, hr_hbm, he_hbm, edge_hbm, tmp_hbm, htail_hbm,
               acc_out, den_out, gmax_out,
               hl_t, hr_t, he_t, head_t, tail_t, tmp_t, ex_t, rows_t,
               den_t, iota_t, mbuf, gall,
               acc_sh, den_sh, gmax_sh,
               semg0, semg1, semg2, sems0, sems1, sems2):
    cid = lax.axis_index("c")
    sid = lax.axis_index("s")
    w = cid * NT + sid

    stage = [
        pltpu.async_copy(hl_hbm, hl_t, semg0),
        pltpu.async_copy(hr_hbm, hr_t, semg0),
        pltpu.async_copy(he_hbm, he_t, semg0),
        pltpu.async_copy(edge_hbm.at[0, pl.ds(w * EPT, EPT)], head_t, semg1),
        pltpu.async_copy(edge_hbm.at[1, pl.ds(w * EPT, EPT)], tail_t, semg1),
        pltpu.async_copy(tmp_hbm.at[pl.ds(w * EPT, EPT)], tmp_t, semg1),
    ]

    zv = jnp.zeros((16,), jnp.float32)
    lanes = lax.iota(jnp.int32, 16)

    def zden(i, c):
        for v in range(4):
            den_t[i, pl.ds(v * 16, 16)] = zv
        return c

    lax.fori_loop(0, DRW, zden, 0)
    for v in range(GRP // 16):
        iota_t[0, pl.ds(v * 16, 16)] = lanes + v * 16
        iota_t[1, pl.ds(v * 16, 16)] = lanes + (GRP + v * 16)

    zero = [
        pltpu.async_copy(den_t.at[pl.ds(0, 125)],
                         acc_sh.at[pl.ds(sid * RPT + k * 125, 125)], semg2)
        for k in range(5)
    ]
    zero.append(
        pltpu.async_copy(den_t.at[pl.ds(0, DRW // NT)],
                         den_sh.at[pl.ds(sid * (DRW // NT), DRW // NT)],
                         semg2))
    for c in stage:
        c.wait()

    def p1(g, m):
        for v in range(GRP // 16):
            sl = pl.ds(g * GRP + v * 16, 16)
            hi = head_t[sl]
            ti = tail_t[sl]
            ei = tmp_t[sl]
            s = (plsc.load_gather(hl_t, [hi])
                 + plsc.load_gather(hr_t, [ti])
                 + plsc.load_gather(he_t, [ei]))
            s = jnp.where(s >= 0.0, s, 0.2 * s)
            ex_t[sl] = s
            m = jnp.maximum(m, s)
        return m

    m = lax.fori_loop(0, NG, p1, jnp.full((16,), -3.0e38, jnp.float32))

    mbuf[...] = jnp.full((16,), jnp.max(m), jnp.float32)
    for c in zero:
        c.wait()
    pltpu.sync_copy(mbuf, gmax_sh.at[sid])
    plsc.subcore_barrier()
    pltpu.sync_copy(gmax_sh, gall)
    gv = gall[0, :]
    for t in range(1, 16):
        gv = jnp.maximum(gv, gall[t, :])
    mbuf[...] = gv

    @pl.when(sid == 0)
    def _():
        pltpu.sync_copy(mbuf, gmax_out.at[cid, 0])

    def p1c(g, c):
        for v in range(GRP // 16):
            sl = pl.ds(g * GRP + v * 16, 16)
            ex = jnp.exp(ex_t[sl] - gv)
            ex_t[sl] = ex
            hv = head_t[sl]
            hi = lax.shift_right_logical(hv, 6)
            lo = lax.bitwise_and(hv, 63)
            for j in range(16):
                plsc.addupdate_scatter(
                    den_t, [hi, lo], ex, mask=lanes == j)
        return c

    lax.fori_loop(0, NG, p1c, 0)

    for k in range(2):
        pltpu.sync_copy(den_t.at[pl.ds(k * GRP, GRP)],
                        den_sh.at[iota_t.at[k]], add=True)

    sems_g = (semg0, semg1, semg2)
    sems_s = (sems0, sems1, sems2)

    def start_gather(g, b):
        pltpu.async_copy(htail_hbm.at[tail_t.at[pl.ds(g * GRP, GRP)]],
                         rows_t.at[b], sems_g[b])

    def wait_gather(g, b):
        pltpu.make_async_copy(
            htail_hbm.at[tail_t.at[pl.ds(g * GRP, GRP)]],
            rows_t.at[b], sems_g[b]).wait()

    def start_scatter(g, b):
        pltpu.async_copy(
            rows_t.at[b], acc_sh.at[head_t.at[pl.ds(g * GRP, GRP)]],
            sems_s[b], add=True)

    def wait_scatter(g, b):
        pltpu.make_async_copy(
            rows_t.at[b], acc_sh.at[head_t.at[pl.ds(g * GRP, GRP)]],
            sems_s[b]).wait()

    def scale(g, b):
        for k in range(GRP // 16):
            exg = ex_t[pl.ds(g * GRP + k * 16, 16)]
            sps = [
                exg.at[jnp.full((16,), j, jnp.int32)].get(
                    mode="promise_in_bounds")
                for j in range(16)
            ]
            for j in range(16):
                r = k * 16 + j
                for v in range(D_OUT // 16):
                    sl = pl.ds(v * 16, 16)
                    rows_t[b, r, sl] = rows_t[b, r, sl] * sps[j]

    def step(g, b, nb):
        wait_gather(g, b)
        scale(g, b)
        start_scatter(g, b)

        @pl.when(g + 2 < NG)
        def _():
            @pl.when(g >= 1)
            def _():
                wait_scatter(g - 1, nb)
            start_gather(g + 2, nb)

    start_gather(0, 0)
    start_gather(1, 1)

    def p2(g, c):
        gm = lax.rem(g, 3)
        for b in range(NBUF):
            @pl.when(gm == b)
            def _(b=b):
                step(g, b, (b + 2) % 3)
        return c

    lax.fori_loop(0, NG, p2, 0)
    for q in range(NBUF):
        gq = NG - NBUF + q
        wait_scatter(gq, gq % 3)

    plsc.subcore_barrier()
    pltpu.sync_copy(acc_sh.at[pl.ds(sid * RPT, RPT)],
                    acc_out.at[cid, pl.ds(sid * RPT, RPT)])
    pltpu.sync_copy(den_sh.at[pl.ds(sid * (DRW // NT), DRW // NT)],
                    den_out.at[cid, pl.ds(sid * (DRW // NT), DRW // NT)])


def _post_body(acc_ref, den_ref, s_ref, res_ref, out_ref):
    num = acc_ref[0] * s_ref[0] + acc_ref[1] * s_ref[1]
    den = den_ref[0, 0:N] * s_ref[0] + den_ref[1, 0:N] * s_ref[1]
    recip = jnp.where(den > 0.0, 1.0 / den, 0.0)
    out_ref[...] = num * recip + res_ref[...]


_post_call = pl.pallas_call(
    _post_body,
    in_specs=(
        pl.BlockSpec(memory_space=pltpu.VMEM),
        pl.BlockSpec(memory_space=pltpu.VMEM),
        pl.BlockSpec(memory_space=pltpu.SMEM),
        pl.BlockSpec(memory_space=pltpu.VMEM),
    ),
    out_shape=jax.ShapeDtypeStruct((N, D_OUT), jnp.float32),
)


def kernel(head_feature, tail_feature, edge_index, tmp_edge, W, W_e,
           a_l, a_r, a_e, edge_emb, W_res, b_res):
    al = a_l.reshape(D_OUT, 1)
    ar = a_r.reshape(D_OUT, 1)
    ae = a_e.reshape(1, 64)
    bres2 = b_res.reshape(1, D_OUT)

    htail, hlr, he_out, res = _pre_call(
        head_feature, tail_feature, W, al, ar, edge_emb, W_e, ae, W_res, bres2)

    hl = hlr[:, 0]
    hr = hlr[:, 1]
    he16 = he_out.reshape(16)

    acc, den, gmax = _sc_kernel(hl, hr, he16, edge_index, tmp_edge, htail)

    g = jnp.max(gmax[:, 0, :], axis=1)
    s2 = jnp.exp(g - jnp.max(g))
    denP = den.reshape(NSC, DRW * 64, 1)
    return _post_call(acc, denP, s2, res)


# speedup vs baseline: 1.1166x; 1.0070x over previous
import functools

import jax
import jax.numpy as jnp
from jax import lax
from jax.experimental import pallas as pl
from jax.experimental.pallas import tpu as pltpu
from jax.experimental.pallas import tpu_sc as plsc

N = 10000
E = 320000
D_IN = 128
D_OUT = 64
NT = 16
NSC = 2
NW = NT * NSC
EPT = E // NW
GRP = 80
NG = EPT // GRP
RPT = N // NT
DRW = 160
NBUF = 3


def _pre_body(head_ref, tail_ref, w_ref, al_ref, ar_ref, emb_ref, we_ref,
              ae_ref, wres_ref, bres_ref,
              htail_ref, hlr_ref, he_ref, res_ref):
    w = w_ref[...]
    hh = jnp.dot(head_ref[...], w, preferred_element_type=jnp.float32)
    ht = jnp.dot(tail_ref[...], w, preferred_element_type=jnp.float32)
    htail_ref[...] = ht
    hlr_ref[:, 0:1] = jnp.dot(hh, al_ref[...], preferred_element_type=jnp.float32)
    hlr_ref[:, 1:2] = jnp.dot(ht, ar_ref[...], preferred_element_type=jnp.float32)
    e5 = jnp.dot(emb_ref[...], we_ref[...], preferred_element_type=jnp.float32)
    he5 = jnp.sum(e5 * ae_ref[...], axis=1, keepdims=True)
    he_ref[...] = jnp.concatenate(
        [he5, jnp.zeros((11, 1), jnp.float32)], axis=0)
    res_ref[...] = (
        jnp.dot(head_ref[...], wres_ref[...], preferred_element_type=jnp.float32)
        + bres_ref[...])


_pre_call = pl.pallas_call(
    _pre_body,
    out_shape=(
        jax.ShapeDtypeStruct((N, D_OUT), jnp.float32),
        jax.ShapeDtypeStruct((N, 2), jnp.float32),
        jax.ShapeDtypeStruct((16, 1), jnp.float32),
        jax.ShapeDtypeStruct((N, D_OUT), jnp.float32),
    ),
)


_mesh = plsc.VectorSubcoreMesh(core_axis_name="c", subcore_axis_name="s")


@functools.partial(
    pl.kernel,
    out_type=(
        jax.ShapeDtypeStruct((NSC, N, D_OUT), jnp.float32),
        jax.ShapeDtypeStruct((NSC, DRW, 64), jnp.float32),
        jax.ShapeDtypeStruct((NSC, 8, 16), jnp.float32),
    ),
    mesh=_mesh,
    compiler_params=pltpu.CompilerParams(
        needs_layout_passes=False, use_tc_tiling_on_sc=False),
    scratch_types=(
        pltpu.VMEM((N,), jnp.float32),
        pltpu.VMEM((N,), jnp.float32),
        pltpu.VMEM((16,), jnp.float32),
        pltpu.VMEM((EPT,), jnp.int32),
        pltpu.VMEM((EPT,), jnp.int32),
        pltpu.VMEM((EPT,), jnp.int32),
        pltpu.VMEM((EPT,), jnp.float32),
        pltpu.VMEM((NBUF, GRP, D_OUT), jnp.float32),
        pltpu.VMEM((DRW, 64), jnp.float32),
        pltpu.VMEM((16,), jnp.float32),
        pltpu.VMEM((16, 16), jnp.float32),
        pltpu.VMEM_SHARED((N, D_OUT), jnp.float32),
        pltpu.VMEM_SHARED((DRW, 64), jnp.float32),
        pltpu.VMEM_SHARED((16, 16), jnp.float32),
        pltpu.SemaphoreType.DMA,
        pltpu.SemaphoreType.DMA,
        pltpu.SemaphoreType.DMA,
        pltpu.SemaphoreType.DMA,
        pltpu.SemaphoreType.DMA,
        pltpu.SemaphoreType.DMA,
    ),
)
def _sc_kernel(hl_hbm, hr_hbm, he_hbm, edge_hbm, tmp_hbm, htail_hbm,
               acc_out, den_out, gmax_out,
               hl_t, hr_t, he_t, head_t, tail_t, tmp_t, ex_t, rows_t,
               den_t, mbuf, gall,
               acc_sh, den_sh, gmax_sh,
               semg0, semg1, semg2, sems0, sems1, sems2):
    cid = lax.axis_index("c")
    sid = lax.axis_index("s")
    w = cid * NT + sid

    stage = [
        pltpu.async_copy(hl_hbm, hl_t, semg0),
        pltpu.async_copy(hr_hbm, hr_t, semg0),
        pltpu.async_copy(he_hbm, he_t, semg0),
        pltpu.async_copy(edge_hbm.at[0, pl.ds(w * EPT, EPT)], head_t, semg1),
        pltpu.async_copy(edge_hbm.at[1, pl.ds(w * EPT, EPT)], tail_t, semg1),
        pltpu.async_copy(tmp_hbm.at[pl.ds(w * EPT, EPT)], tmp_t, semg1),
    ]

    zv = jnp.zeros((16,), jnp.float32)
    lanes = lax.iota(jnp.int32, 16)

    def zden(i, c):
        for v in range(4):
            den_t[i, pl.ds(v * 16, 16)] = zv
        return c

    lax.fori_loop(0, DRW, zden, 0)

    zero = [
        pltpu.async_copy(den_t.at[pl.ds(0, 125)],
                         acc_sh.at[pl.ds(sid * RPT + k * 125, 125)], semg2)
        for k in range(5)
    ]
    zero.append(
        pltpu.async_copy(den_t.at[pl.ds(0, DRW // NT)],
                         den_sh.at[pl.ds(sid * (DRW // NT), DRW // NT)],
                         semg2))
    for c in stage:
        c.wait()

    for _b, _sem in ((0, semg0), (1, semg1)):
        pltpu.async_copy(htail_hbm.at[tail_t.at[pl.ds(_b * GRP, GRP)]],
                         rows_t.at[_b], _sem)

    def p1(g, m):
        for v in range(GRP // 16):
            sl = pl.ds(g * GRP + v * 16, 16)
            hi = head_t[sl]
            ti = tail_t[sl]
            ei = tmp_t[sl]
            s = (plsc.load_gather(hl_t, [hi])
                 + plsc.load_gather(hr_t, [ti])
                 + plsc.load_gather(he_t, [ei]))
            s = jnp.where(s >= 0.0, s, 0.2 * s)
            ex_t[sl] = s
            m = jnp.maximum(m, s)
        return m

    m = lax.fori_loop(0, NG, p1, jnp.full((16,), -3.0e38, jnp.float32))

    for v in range(2 * GRP // 16):
        tmp_t[pl.ds(v * 16, 16)] = lanes + v * 16

    mbuf[...] = jnp.full((16,), jnp.max(m), jnp.float32)
    for c in zero:
        c.wait()
    pltpu.sync_copy(mbuf, gmax_sh.at[sid])
    plsc.subcore_barrier()
    pltpu.sync_copy(gmax_sh, gall)
    gv = gall[0, :]
    for t in range(1, 16):
        gv = jnp.maximum(gv, gall[t, :])
    mbuf[...] = gv

    @pl.when(sid == 0)
    def _():
        pltpu.sync_copy(mbuf, gmax_out.at[cid, 0])

    def p1c(g, c):
        for v in range(GRP // 16):
            sl = pl.ds(g * GRP + v * 16, 16)
            ex = jnp.exp(ex_t[sl] - gv)
            ex_t[sl] = ex
            hv = head_t[sl]
            hi = lax.shift_right_logical(hv, 6)
            lo = lax.bitwise_and(hv, 63)
            for j in range(16):
                plsc.addupdate_scatter(
                    den_t, [hi, lo], ex, mask=lanes == j)
        return c

    lax.fori_loop(0, NG, p1c, 0)

    for k in range(2):
        pltpu.sync_copy(den_t.at[pl.ds(k * GRP, GRP)],
                        den_sh.at[tmp_t.at[pl.ds(k * GRP, GRP)]], add=True)

    sems_g = (semg0, semg1, semg2)
    sems_s = (sems0, sems1, sems2)

    def start_gather(g, b):
        pltpu.async_copy(htail_hbm.at[tail_t.at[pl.ds(g * GRP, GRP)]],
                         rows_t.at[b], sems_g[b])

    def wait_gather(g, b):
        pltpu.make_async_copy(
            htail_hbm.at[tail_t.at[pl.ds(g * GRP, GRP)]],
            rows_t.at[b], sems_g[b]).wait()

    def start_scatter(g, b):
        pltpu.async_copy(
            rows_t.at[b], acc_sh.at[head_t.at[pl.ds(g * GRP, GRP)]],
            sems_s[b], add=True)

    def wait_scatter(g, b):
        pltpu.make_async_copy(
            rows_t.at[b], acc_sh.at[head_t.at[pl.ds(g * GRP, GRP)]],
            sems_s[b]).wait()

    def scale(g, b):
        for k in range(GRP // 16):
            exg = ex_t[pl.ds(g * GRP + k * 16, 16)]
            sps = [
                exg.at[jnp.full((16,), j, jnp.int32)].get(
                    mode="promise_in_bounds")
                for j in range(16)
            ]
            for j in range(16):
                r = k * 16 + j
                for v in range(D_OUT // 16):
                    sl = pl.ds(v * 16, 16)
                    rows_t[b, r, sl] = rows_t[b, r, sl] * sps[j]

    def step(g, b, nb):
        wait_gather(g, b)
        scale(g, b)
        start_scatter(g, b)

        @pl.when(g + 2 < NG)
        def _():
            @pl.when(g >= 1)
            def _():
                wait_scatter(g - 1, nb)
            start_gather(g + 2, nb)

    def p2(g, c):
        gm = lax.rem(g, 3)
        for b in range(NBUF):
            @pl.when(gm == b)
            def _(b=b):
                step(g, b, (b + 2) % 3)
        return c

    lax.fori_loop(0, NG, p2, 0)
    for q in range(NBUF):
        gq = NG - NBUF + q
        wait_scatter(gq, gq % 3)

    plsc.subcore_barrier()
    pltpu.sync_copy(acc_sh.at[pl.ds(sid * RPT, RPT)],
                    acc_out.at[cid, pl.ds(sid * RPT, RPT)])
    pltpu.sync_copy(den_sh.at[pl.ds(sid * (DRW // NT), DRW // NT)],
                    den_out.at[cid, pl.ds(sid * (DRW // NT), DRW // NT)])


def _post_body(acc_ref, den_ref, s_ref, res_ref, out_ref):
    num = acc_ref[0] * s_ref[0] + acc_ref[1] * s_ref[1]
    den = den_ref[0, 0:N] * s_ref[0] + den_ref[1, 0:N] * s_ref[1]
    recip = jnp.where(den > 0.0, 1.0 / den, 0.0)
    out_ref[...] = num * recip + res_ref[...]


_post_call = pl.pallas_call(
    _post_body,
    in_specs=(
        pl.BlockSpec(memory_space=pltpu.VMEM),
        pl.BlockSpec(memory_space=pltpu.VMEM),
        pl.BlockSpec(memory_space=pltpu.SMEM),
        pl.BlockSpec(memory_space=pltpu.VMEM),
    ),
    out_shape=jax.ShapeDtypeStruct((N, D_OUT), jnp.float32),
)


def kernel(head_feature, tail_feature, edge_index, tmp_edge, W, W_e,
           a_l, a_r, a_e, edge_emb, W_res, b_res):
    al = a_l.reshape(D_OUT, 1)
    ar = a_r.reshape(D_OUT, 1)
    ae = a_e.reshape(1, 64)
    bres2 = b_res.reshape(1, D_OUT)

    htail, hlr, he_out, res = _pre_call(
        head_feature, tail_feature, W, al, ar, edge_emb, W_e, ae, W_res, bres2)

    hl = hlr[:, 0]
    hr = hlr[:, 1]
    he16 = he_out.reshape(16)

    acc, den, gmax = _sc_kernel(hl, hr, he16, edge_index, tmp_edge, htail)

    g = jnp.max(gmax[:, 0, :], axis=1)
    s2 = jnp.exp(g - jnp.max(g))
    denP = den.reshape(NSC, DRW * 64, 1)
    return _post_call(acc, denP, s2, res)
